# Initial kernel scaffold; baseline (speedup 1.0000x reference)
import jax
import jax.numpy as jnp
from jax.experimental import pallas as pl


def kernel(intra, node_repr, edge_index, edge_type, rel_emb):
    raise NotImplementedError("write your pallas kernel here")


# speedup vs baseline: 5.3271x; 5.3271x over previous
import jax
import jax.numpy as jnp
from jax import lax
from jax.experimental import pallas as pl
from jax.experimental.pallas import tpu as pltpu
from jax.experimental.pallas import tpu_sc as plsc

N_EDGES = 320000
D = 128
N_SEG = 5
CHUNK = 128
N_CHUNKS = N_EDGES // CHUNK
NC, NS = 2, 16
NW = NC * NS
ITERS = -(-N_CHUNKS // NW)


def _body(intra_h, node_h, src_h, dst_h, et_h, rel_h, out_h,
          src_v, dst_v, et_v, asm_v, sem):
    wid = lax.axis_index("s") * NC + lax.axis_index("c")

    def chunk_body(i, carry):
        c = wid + i * NW

        @pl.when(c < N_CHUNKS)
        def _():
            base = c * CHUNK
            pltpu.sync_copy(src_h.at[pl.ds(base, CHUNK)], src_v)
            pltpu.sync_copy(dst_h.at[pl.ds(base, CHUNK)], dst_v)
            pltpu.sync_copy(et_h.at[pl.ds(base, CHUNK)], et_v)
            c0 = pltpu.async_copy(intra_h.at[src_v], asm_v.at[:, pl.ds(0 * D, D)], sem)
            c1 = pltpu.async_copy(intra_h.at[dst_v], asm_v.at[:, pl.ds(1 * D, D)], sem)
            c2 = pltpu.async_copy(node_h.at[src_v], asm_v.at[:, pl.ds(2 * D, D)], sem)
            c3 = pltpu.async_copy(node_h.at[dst_v], asm_v.at[:, pl.ds(3 * D, D)], sem)
            c4 = pltpu.async_copy(rel_h.at[et_v], asm_v.at[:, pl.ds(4 * D, D)], sem)
            c0.wait(); c1.wait(); c2.wait(); c3.wait(); c4.wait()
            pltpu.sync_copy(asm_v, out_h.at[pl.ds(base, CHUNK)])

        return carry

    lax.fori_loop(0, ITERS, chunk_body, None)


_gather_concat = pl.kernel(
    _body,
    out_type=jax.ShapeDtypeStruct((N_EDGES, N_SEG * D), jnp.float32),
    mesh=plsc.VectorSubcoreMesh(core_axis_name="c", subcore_axis_name="s"),
    scratch_types=[
        pltpu.VMEM((CHUNK,), jnp.int32),
        pltpu.VMEM((CHUNK,), jnp.int32),
        pltpu.VMEM((CHUNK,), jnp.int32),
        pltpu.VMEM((CHUNK, N_SEG * D), jnp.float32),
        pltpu.SemaphoreType.DMA,
    ],
)


@jax.jit
def kernel(intra, node_repr, edge_index, edge_type, rel_emb):
    src = edge_index[0].astype(jnp.int32)
    dst = edge_index[1].astype(jnp.int32)
    et = edge_type.astype(jnp.int32)
    return _gather_concat(intra, node_repr, src, dst, et, rel_emb)


# speedup vs baseline: 5.7351x; 1.0766x over previous
import jax
import jax.numpy as jnp
from jax import lax
from jax.experimental import pallas as pl
from jax.experimental.pallas import tpu as pltpu
from jax.experimental.pallas import tpu_sc as plsc

N_EDGES = 320000
D = 128
N_SEG = 5
NC, NS = 2, 16
NW = NC * NS
CHUNK = 80
CPW = N_EDGES // CHUNK // NW
EPW = CHUNK * CPW
IBLK = 25
IB_EDGES = IBLK * CHUNK


def _body(intra_h, node_h, src_h, dst_h, et_h, rel_h, out_h,
          src_v, dst_v, et_v, asm0, asm1, gsem, wsem0, wsem1):
    wid = lax.axis_index("s") * NC + lax.axis_index("c")
    e0 = wid * EPW
    asms = (asm0, asm1)
    wsems = (wsem0, wsem1)

    def outer(i, carry):
        for b in range(2):
            j = 2 * i + b

            @pl.when(j < CPW)
            def _():
                @pl.when(j % IBLK == 0)
                def _():
                    off = e0 + (j // IBLK) * IB_EDGES
                    pltpu.sync_copy(src_h.at[pl.ds(off, IB_EDGES)], src_v)
                    pltpu.sync_copy(dst_h.at[pl.ds(off, IB_EDGES)], dst_v)
                    pltpu.sync_copy(et_h.at[pl.ds(off, IB_EDGES)], et_v)

                @pl.when(j >= 2)
                def _():
                    pltpu.make_async_copy(
                        asms[b], out_h.at[pl.ds(0, CHUNK)], wsems[b]).wait()

                ioff = (j % IBLK) * CHUNK
                a = asms[b]
                si = src_v.at[pl.ds(ioff, CHUNK)]
                di = dst_v.at[pl.ds(ioff, CHUNK)]
                ti = et_v.at[pl.ds(ioff, CHUNK)]
                c0 = pltpu.async_copy(intra_h.at[si], a.at[:, pl.ds(0 * D, D)], gsem)
                c1 = pltpu.async_copy(intra_h.at[di], a.at[:, pl.ds(1 * D, D)], gsem)
                c2 = pltpu.async_copy(node_h.at[si], a.at[:, pl.ds(2 * D, D)], gsem)
                c3 = pltpu.async_copy(node_h.at[di], a.at[:, pl.ds(3 * D, D)], gsem)
                c4 = pltpu.async_copy(rel_h.at[ti], a.at[:, pl.ds(4 * D, D)], gsem)
                c0.wait(); c1.wait(); c2.wait(); c3.wait(); c4.wait()
                pltpu.async_copy(a, out_h.at[pl.ds(e0 + j * CHUNK, CHUNK)], wsems[b])

        return carry

    lax.fori_loop(0, (CPW + 1) // 2, outer, None)
    pltpu.make_async_copy(asm0, out_h.at[pl.ds(0, CHUNK)], wsem0).wait()
    pltpu.make_async_copy(asm1, out_h.at[pl.ds(0, CHUNK)], wsem1).wait()


_gather_concat = pl.kernel(
    _body,
    out_type=jax.ShapeDtypeStruct((N_EDGES, N_SEG * D), jnp.float32),
    mesh=plsc.VectorSubcoreMesh(core_axis_name="c", subcore_axis_name="s"),
    scratch_types=[
        pltpu.VMEM((IB_EDGES,), jnp.int32),
        pltpu.VMEM((IB_EDGES,), jnp.int32),
        pltpu.VMEM((IB_EDGES,), jnp.int32),
        pltpu.VMEM((CHUNK, N_SEG * D), jnp.float32),
        pltpu.VMEM((CHUNK, N_SEG * D), jnp.float32),
        pltpu.SemaphoreType.DMA,
        pltpu.SemaphoreType.DMA,
        pltpu.SemaphoreType.DMA,
    ],
)


@jax.jit
def kernel(intra, node_repr, edge_index, edge_type, rel_emb):
    src = edge_index[0].astype(jnp.int32)
    dst = edge_index[1].astype(jnp.int32)
    et = edge_type.astype(jnp.int32)
    return _gather_concat(intra, node_repr, src, dst, et, rel_emb)
